# baseline (speedup 1.0000x reference)
import functools

import jax
import jax.numpy as jnp
from jax.experimental import pallas as pl
from jax.experimental.pallas import tpu as pltpu
from jax.experimental.pallas import tpu_sc as plsc

B = 8
K = 64
EMB = 256
F = 23
LB_COL = 21
UB_COL = 22
INF_THRESHOLD = 1e18
THRESH = 10
NCLS = THRESH + 1

BLOCK_N = 1000

SC_NC = 2
SC_NS = 16
SC_NW = SC_NC * SC_NS
SC_LANES = 16
BPW = 320


def _mask_body(vt_hbm, lb_hbm, ub_hbm, mb_hbm, ms_hbm, ml_hbm, rg_hbm,
               vt_v, lb_v, ub_v, mb_v, ms_v, ml_v, rg_v):
    n = vt_hbm.shape[0]
    wid = jax.lax.axis_index("s") * SC_NC + jax.lax.axis_index("c")
    base = wid * BPW
    tail = n - (SC_NW - 1) * BPW

    one = jnp.full((SC_LANES,), 1.0, jnp.float32)
    zero = jnp.zeros((SC_LANES,), jnp.float32)

    def body(j, carry):
        sl = pl.ds(j * SC_LANES, SC_LANES)
        vt = vt_v[sl].astype(jnp.float32)
        lb = lb_v[sl]
        ub = ub_v[sl]
        int_f = jnp.where(vt == 2.0, one, zero)
        fin_lb = jnp.where(jnp.abs(lb) < INF_THRESHOLD, one, zero)
        fin_ub = jnp.where(jnp.abs(ub) < INF_THRESHOLD, one, zero)
        rng_f = jnp.where(ub - lb <= THRESH, one, zero)
        small = int_f * fin_lb * fin_ub * rng_f
        mb_v[sl] = jnp.where(vt == 1.0, one, zero)
        ms_v[sl] = small
        ml_v[sl] = int_f * (one - small)
        tu = ub.astype(jnp.int32).astype(jnp.float32)
        cu = tu + jnp.where(tu < ub, one, zero)
        tl = lb.astype(jnp.int32).astype(jnp.float32)
        fl = tl - jnp.where(tl > lb, one, zero)
        rg_v[sl] = jnp.clip(cu - fl + 1.0, 1.0, float(NCLS)).astype(jnp.int32)
        return carry

    def run(nrows):
        def inner():
            sl = pl.ds(0, nrows)
            hsl = pl.ds(base, nrows)
            pltpu.sync_copy(vt_hbm.at[hsl], vt_v.at[sl])
            pltpu.sync_copy(lb_hbm.at[hsl], lb_v.at[sl])
            pltpu.sync_copy(ub_hbm.at[hsl], ub_v.at[sl])
            jax.lax.fori_loop(0, nrows // SC_LANES, body, 0)
            pltpu.sync_copy(mb_v.at[sl], mb_hbm.at[hsl])
            pltpu.sync_copy(ms_v.at[sl], ms_hbm.at[hsl])
            pltpu.sync_copy(ml_v.at[sl], ml_hbm.at[hsl])
            pltpu.sync_copy(rg_v.at[sl], rg_hbm.at[hsl])
        return inner

    pl.when(wid < SC_NW - 1)(run(BPW))
    pl.when(wid == SC_NW - 1)(run(tail))


@functools.cache
def _routing_masks_fn(n):
    return pl.kernel(
        _mask_body,
        out_type=[jax.ShapeDtypeStruct((n,), jnp.float32),
                  jax.ShapeDtypeStruct((n,), jnp.float32),
                  jax.ShapeDtypeStruct((n,), jnp.float32),
                  jax.ShapeDtypeStruct((n,), jnp.int32)],
        mesh=plsc.VectorSubcoreMesh(core_axis_name="c", subcore_axis_name="s",
                                    num_cores=SC_NC, num_subcores=SC_NS),
        scratch_types=[pltpu.VMEM((BPW,), jnp.int32),
                       pltpu.VMEM((BPW,), jnp.float32),
                       pltpu.VMEM((BPW,), jnp.float32),
                       pltpu.VMEM((BPW,), jnp.float32),
                       pltpu.VMEM((BPW,), jnp.float32),
                       pltpu.VMEM((BPW,), jnp.float32),
                       pltpu.VMEM((BPW,), jnp.int32)],
    )


def _norm(x):
    xb = x.astype(jnp.bfloat16)
    ones = jnp.full((EMB, 1), 1.0 / EMB, jnp.bfloat16)
    m = jnp.dot(xb, ones, preferred_element_type=jnp.float32)
    q = jnp.dot(xb * xb, ones, preferred_element_type=jnp.float32)
    v = q - m * m
    return (x - m) * jax.lax.rsqrt(v + 1e-5)


def _gelu(x):
    c1 = jnp.bfloat16(1.5957692)
    c2 = jnp.bfloat16(1.5957692 * 0.044715)
    return x * jax.nn.sigmoid(x * (c1 + c2 * x * x))


def _bdot(a, b):
    return jnp.dot(a, b, preferred_element_type=jnp.float32)


def _fused_kernel(
    tokens_ref, attn_ref, vb_ref, colb_ref, z0_ref,
    mbin_ref, msmall_ref, mlarge_ref, rg_ref,
    dw_ref, db_ref, fg_ref, fb_ref,
    bin_ng, bin_nb, bin_w1, bin_b1, bin_w2, bin_b2, bin_wh, bin_bh,
    int_ng, int_nb, int_w1, int_b1, int_w2, int_b2, int_wh, int_bh,
    lrg_ng, lrg_nb, lrg_w1, lrg_b1, lrg_w2, lrg_b2, lrg_wh, lrg_bh,
    zout_ref, pbin_ref, lsmall_ref, plarge_ref,
    tw_ref, w1c_ref, b1c_ref, w2c_ref, b2c_ref, whc_ref,
):
    @pl.when(pl.program_id(0) == 0)
    def _prep():
        tw_ref[...] = jnp.dot(tokens_ref[...], dw_ref[...],
                              preferred_element_type=jnp.float32).astype(jnp.bfloat16)
        for i, (ng, nb, w1, b1, w2, b2, wh) in enumerate((
                (bin_ng, bin_nb, bin_w1, bin_b1, bin_w2, bin_b2, bin_wh),
                (int_ng, int_nb, int_w1, int_b1, int_w2, int_b2, int_wh),
                (lrg_ng, lrg_nb, lrg_w1, lrg_b1, lrg_w2, lrg_b2, lrg_wh))):
            w1c_ref[:, i * EMB:(i + 1) * EMB] = (
                ng[...][:, None] * w1[...]).astype(jnp.bfloat16)
            b1c_ref[0, i * EMB:(i + 1) * EMB] = (
                jnp.dot(nb[...][None, :], w1[...],
                        preferred_element_type=jnp.float32)[0]
                + b1[...]).astype(jnp.bfloat16)
            w2c_ref[:, i * EMB:(i + 1) * EMB] = w2[...].astype(jnp.bfloat16)
            b2c_ref[0, i * EMB:(i + 1) * EMB] = b2[...].astype(jnp.bfloat16)
            whc_ref[:, i * 128:i * 128 + wh.shape[1]] = wh[...].astype(jnp.bfloat16)

    attn = attn_ref[...].astype(jnp.bfloat16)
    vb = vb_ref[...]
    attn_tiled = jnp.concatenate([attn] * B, axis=1)
    P = jnp.where(colb_ref[...] == vb, attn_tiled, jnp.bfloat16(0.0))
    z = _bdot(P, tw_ref[...]) + db_ref[...]
    z_out = _norm(z + z0_ref[...]) * fg_ref[...] + fb_ref[...]
    zout_ref[...] = z_out

    nz = _norm(z_out)
    z_out_b = z_out.astype(jnp.bfloat16)
    h1 = _bdot(nz.astype(jnp.bfloat16), w1c_ref[...]).astype(jnp.bfloat16) + b1c_ref[...]
    g1 = _gelu(h1)

    def tail(i, out_dim, bh):
        h2 = (_bdot(g1[:, i * EMB:(i + 1) * EMB],
                    w2c_ref[:, i * EMB:(i + 1) * EMB]).astype(jnp.bfloat16)
              + b2c_ref[0, i * EMB:(i + 1) * EMB])
        hr = z_out_b + _gelu(h2)
        return _bdot(hr, whc_ref[:, i * 128:i * 128 + out_dim]) + bh[...]

    out_bin = tail(0, 1, bin_bh)
    pbin_ref[...] = jax.nn.sigmoid(out_bin) * mbin_ref[...]

    logits = tail(1, NCLS, int_bh)
    valid = jax.lax.broadcasted_iota(jnp.int32, (BLOCK_N, NCLS), 1) < rg_ref[...]
    lsmall_ref[...] = msmall_ref[...] * jnp.where(valid, logits, -1e9)

    out_lrg = tail(2, 1, lrg_bh)
    plarge_ref[...] = out_lrg * mlarge_ref[...]


def _row(i):
    return (i, 0)


def _full(i):
    return (0, 0)


def _full1(i):
    return (0,)


@jax.jit
def kernel(evolved_tokens, token_batch, attn_weights, var_types, z_var_0,
           var_batch, variable_features, params):
    n = attn_weights.shape[0]
    grid = (n // BLOCK_N,)
    vb2 = var_batch.astype(jnp.int32)[:, None]

    mbin, msmall, mlarge, ranges = _routing_masks_fn(n)(
        var_types.astype(jnp.int32),
        variable_features[:, LB_COL], variable_features[:, UB_COL])
    mbin = mbin[:, None]
    msmall = msmall[:, None]
    mlarge = mlarge[:, None]
    ranges = ranges[:, None]

    colb = (jnp.arange(B * K, dtype=jnp.int32) // K)[None, :]

    def head_specs(out_dim):
        return [
            pl.BlockSpec((EMB,), _full1),
            pl.BlockSpec((EMB,), _full1),
            pl.BlockSpec((EMB, EMB), _full),
            pl.BlockSpec((EMB,), _full1),
            pl.BlockSpec((EMB, EMB), _full),
            pl.BlockSpec((EMB,), _full1),
            pl.BlockSpec((EMB, out_dim), _full),
            pl.BlockSpec((out_dim,), _full1),
        ]

    def head_args(p):
        return [p['ng'], p['nb'], p['w1'], p['b1'],
                p['w2'], p['b2'], p['wh'], p['bh']]

    in_specs = [
        pl.BlockSpec((B * K, EMB), _full),
        pl.BlockSpec((BLOCK_N, K), _row),
        pl.BlockSpec((BLOCK_N, 1), _row),
        pl.BlockSpec((1, B * K), _full),
        pl.BlockSpec((BLOCK_N, EMB), _row),
        pl.BlockSpec((BLOCK_N, 1), _row),
        pl.BlockSpec((BLOCK_N, 1), _row),
        pl.BlockSpec((BLOCK_N, 1), _row),
        pl.BlockSpec((BLOCK_N, 1), _row),
        pl.BlockSpec((EMB, EMB), _full),
        pl.BlockSpec((EMB,), _full1),
        pl.BlockSpec((EMB,), _full1),
        pl.BlockSpec((EMB,), _full1),
    ] + head_specs(1) + head_specs(NCLS) + head_specs(1)

    out_specs = [
        pl.BlockSpec((BLOCK_N, EMB), _row),
        pl.BlockSpec((BLOCK_N, 1), _row),
        pl.BlockSpec((BLOCK_N, NCLS), _row),
        pl.BlockSpec((BLOCK_N, 1), _row),
    ]
    out_shape = [
        jax.ShapeDtypeStruct((n, EMB), jnp.float32),
        jax.ShapeDtypeStruct((n, 1), jnp.float32),
        jax.ShapeDtypeStruct((n, NCLS), jnp.float32),
        jax.ShapeDtypeStruct((n, 1), jnp.float32),
    ]

    args = [evolved_tokens, attn_weights, vb2, colb, z_var_0,
            mbin, msmall, mlarge, ranges,
            params['deslice_w'], params['deslice_b'], params['fus_g'], params['fus_b']]
    args += head_args(params['bin']) + head_args(params['ints']) + head_args(params['intl'])

    z_out, prob_bin, logits_int_small, pred_int_large = pl.pallas_call(
        _fused_kernel,
        grid=grid,
        in_specs=in_specs,
        out_specs=out_specs,
        out_shape=out_shape,
        scratch_shapes=[
            pltpu.VMEM((B * K, EMB), jnp.bfloat16),
            pltpu.VMEM((EMB, 3 * EMB), jnp.bfloat16),
            pltpu.VMEM((1, 3 * EMB), jnp.bfloat16),
            pltpu.VMEM((EMB, 3 * EMB), jnp.bfloat16),
            pltpu.VMEM((1, 3 * EMB), jnp.bfloat16),
            pltpu.VMEM((EMB, 3 * 128), jnp.bfloat16),
        ],
    )(*args)
    return (z_out, prob_bin, logits_int_small, pred_int_large)


# speedup vs baseline: 1.0058x; 1.0058x over previous
import functools

import jax
import jax.numpy as jnp
from jax.experimental import pallas as pl
from jax.experimental.pallas import tpu as pltpu
from jax.experimental.pallas import tpu_sc as plsc

B = 8
K = 64
EMB = 256
F = 23
LB_COL = 21
UB_COL = 22
INF_THRESHOLD = 1e18
THRESH = 10
NCLS = THRESH + 1

BLOCK_N = 2000

SC_NC = 2
SC_NS = 16
SC_NW = SC_NC * SC_NS
SC_LANES = 16
BPW = 320


def _mask_body(vt_hbm, lb_hbm, ub_hbm, mb_hbm, ms_hbm, ml_hbm, rg_hbm,
               vt_v, lb_v, ub_v, mb_v, ms_v, ml_v, rg_v):
    n = vt_hbm.shape[0]
    wid = jax.lax.axis_index("s") * SC_NC + jax.lax.axis_index("c")
    base = wid * BPW
    tail = n - (SC_NW - 1) * BPW

    one = jnp.full((SC_LANES,), 1.0, jnp.float32)
    zero = jnp.zeros((SC_LANES,), jnp.float32)

    def body(j, carry):
        sl = pl.ds(j * SC_LANES, SC_LANES)
        vt = vt_v[sl].astype(jnp.float32)
        lb = lb_v[sl]
        ub = ub_v[sl]
        int_f = jnp.where(vt == 2.0, one, zero)
        fin_lb = jnp.where(jnp.abs(lb) < INF_THRESHOLD, one, zero)
        fin_ub = jnp.where(jnp.abs(ub) < INF_THRESHOLD, one, zero)
        rng_f = jnp.where(ub - lb <= THRESH, one, zero)
        small = int_f * fin_lb * fin_ub * rng_f
        mb_v[sl] = jnp.where(vt == 1.0, one, zero)
        ms_v[sl] = small
        ml_v[sl] = int_f * (one - small)
        tu = ub.astype(jnp.int32).astype(jnp.float32)
        cu = tu + jnp.where(tu < ub, one, zero)
        tl = lb.astype(jnp.int32).astype(jnp.float32)
        fl = tl - jnp.where(tl > lb, one, zero)
        rg_v[sl] = jnp.clip(cu - fl + 1.0, 1.0, float(NCLS)).astype(jnp.int32)
        return carry

    def run(nrows):
        def inner():
            sl = pl.ds(0, nrows)
            hsl = pl.ds(base, nrows)
            pltpu.sync_copy(vt_hbm.at[hsl], vt_v.at[sl])
            pltpu.sync_copy(lb_hbm.at[hsl], lb_v.at[sl])
            pltpu.sync_copy(ub_hbm.at[hsl], ub_v.at[sl])
            jax.lax.fori_loop(0, nrows // SC_LANES, body, 0)
            pltpu.sync_copy(mb_v.at[sl], mb_hbm.at[hsl])
            pltpu.sync_copy(ms_v.at[sl], ms_hbm.at[hsl])
            pltpu.sync_copy(ml_v.at[sl], ml_hbm.at[hsl])
            pltpu.sync_copy(rg_v.at[sl], rg_hbm.at[hsl])
        return inner

    pl.when(wid < SC_NW - 1)(run(BPW))
    pl.when(wid == SC_NW - 1)(run(tail))


@functools.cache
def _routing_masks_fn(n):
    return pl.kernel(
        _mask_body,
        out_type=[jax.ShapeDtypeStruct((n,), jnp.float32),
                  jax.ShapeDtypeStruct((n,), jnp.float32),
                  jax.ShapeDtypeStruct((n,), jnp.float32),
                  jax.ShapeDtypeStruct((n,), jnp.int32)],
        mesh=plsc.VectorSubcoreMesh(core_axis_name="c", subcore_axis_name="s",
                                    num_cores=SC_NC, num_subcores=SC_NS),
        scratch_types=[pltpu.VMEM((BPW,), jnp.int32),
                       pltpu.VMEM((BPW,), jnp.float32),
                       pltpu.VMEM((BPW,), jnp.float32),
                       pltpu.VMEM((BPW,), jnp.float32),
                       pltpu.VMEM((BPW,), jnp.float32),
                       pltpu.VMEM((BPW,), jnp.float32),
                       pltpu.VMEM((BPW,), jnp.int32)],
    )


def _norm(x):
    xb = x.astype(jnp.bfloat16)
    ones = jnp.full((EMB, 1), 1.0 / EMB, jnp.bfloat16)
    m = jnp.dot(xb, ones, preferred_element_type=jnp.float32)
    q = jnp.dot(xb * xb, ones, preferred_element_type=jnp.float32)
    v = q - m * m
    return (x - m) * jax.lax.rsqrt(v + 1e-5)


def _gelu(x):
    c1 = jnp.bfloat16(1.5957692)
    c2 = jnp.bfloat16(1.5957692 * 0.044715)
    return x * jax.nn.sigmoid(x * (c1 + c2 * x * x))


def _bdot(a, b):
    return jnp.dot(a, b, preferred_element_type=jnp.float32)


def _fused_kernel(
    tokens_ref, attn_ref, vb_ref, colb_ref, z0_ref,
    mbin_ref, msmall_ref, mlarge_ref, rg_ref,
    dw_ref, db_ref, fg_ref, fb_ref,
    bin_ng, bin_nb, bin_w1, bin_b1, bin_w2, bin_b2, bin_wh, bin_bh,
    int_ng, int_nb, int_w1, int_b1, int_w2, int_b2, int_wh, int_bh,
    lrg_ng, lrg_nb, lrg_w1, lrg_b1, lrg_w2, lrg_b2, lrg_wh, lrg_bh,
    zout_ref, pbin_ref, lsmall_ref, plarge_ref,
    tw_ref, w1c_ref, b1c_ref, w2c_ref, b2c_ref, whc_ref,
):
    @pl.when(pl.program_id(0) == 0)
    def _prep():
        tw_ref[...] = jnp.dot(tokens_ref[...], dw_ref[...],
                              preferred_element_type=jnp.float32).astype(jnp.bfloat16)
        for i, (ng, nb, w1, b1, w2, b2, wh) in enumerate((
                (bin_ng, bin_nb, bin_w1, bin_b1, bin_w2, bin_b2, bin_wh),
                (int_ng, int_nb, int_w1, int_b1, int_w2, int_b2, int_wh),
                (lrg_ng, lrg_nb, lrg_w1, lrg_b1, lrg_w2, lrg_b2, lrg_wh))):
            w1c_ref[:, i * EMB:(i + 1) * EMB] = (
                ng[...][:, None] * w1[...]).astype(jnp.bfloat16)
            b1c_ref[0, i * EMB:(i + 1) * EMB] = (
                jnp.dot(nb[...][None, :], w1[...],
                        preferred_element_type=jnp.float32)[0]
                + b1[...]).astype(jnp.bfloat16)
            w2c_ref[:, i * EMB:(i + 1) * EMB] = w2[...].astype(jnp.bfloat16)
            b2c_ref[0, i * EMB:(i + 1) * EMB] = b2[...].astype(jnp.bfloat16)
            whc_ref[:, i * 128:i * 128 + wh.shape[1]] = wh[...].astype(jnp.bfloat16)

    attn = attn_ref[...].astype(jnp.bfloat16)
    vb = vb_ref[...]
    attn_tiled = jnp.concatenate([attn] * B, axis=1)
    P = jnp.where(colb_ref[...] == vb, attn_tiled, jnp.bfloat16(0.0))
    z = _bdot(P, tw_ref[...]) + db_ref[...]
    z_out = _norm(z + z0_ref[...]) * fg_ref[...] + fb_ref[...]
    zout_ref[...] = z_out

    nz = _norm(z_out)
    z_out_b = z_out.astype(jnp.bfloat16)
    h1 = _bdot(nz.astype(jnp.bfloat16), w1c_ref[...]).astype(jnp.bfloat16) + b1c_ref[...]
    g1 = _gelu(h1)

    def tail(i, out_dim, bh):
        h2 = (_bdot(g1[:, i * EMB:(i + 1) * EMB],
                    w2c_ref[:, i * EMB:(i + 1) * EMB]).astype(jnp.bfloat16)
              + b2c_ref[0, i * EMB:(i + 1) * EMB])
        hr = z_out_b + _gelu(h2)
        return _bdot(hr, whc_ref[:, i * 128:i * 128 + out_dim]) + bh[...]

    out_bin = tail(0, 1, bin_bh)
    pbin_ref[...] = jax.nn.sigmoid(out_bin) * mbin_ref[...]

    logits = tail(1, NCLS, int_bh)
    valid = jax.lax.broadcasted_iota(jnp.int32, (BLOCK_N, NCLS), 1) < rg_ref[...]
    lsmall_ref[...] = msmall_ref[...] * jnp.where(valid, logits, -1e9)

    out_lrg = tail(2, 1, lrg_bh)
    plarge_ref[...] = out_lrg * mlarge_ref[...]


def _row(i):
    return (i, 0)


def _full(i):
    return (0, 0)


def _full1(i):
    return (0,)


@jax.jit
def kernel(evolved_tokens, token_batch, attn_weights, var_types, z_var_0,
           var_batch, variable_features, params):
    n = attn_weights.shape[0]
    grid = (n // BLOCK_N,)
    vb2 = var_batch.astype(jnp.int32)[:, None]

    mbin, msmall, mlarge, ranges = _routing_masks_fn(n)(
        var_types.astype(jnp.int32),
        variable_features[:, LB_COL], variable_features[:, UB_COL])
    mbin = mbin[:, None]
    msmall = msmall[:, None]
    mlarge = mlarge[:, None]
    ranges = ranges[:, None]

    colb = (jnp.arange(B * K, dtype=jnp.int32) // K)[None, :]

    def head_specs(out_dim):
        return [
            pl.BlockSpec((EMB,), _full1),
            pl.BlockSpec((EMB,), _full1),
            pl.BlockSpec((EMB, EMB), _full),
            pl.BlockSpec((EMB,), _full1),
            pl.BlockSpec((EMB, EMB), _full),
            pl.BlockSpec((EMB,), _full1),
            pl.BlockSpec((EMB, out_dim), _full),
            pl.BlockSpec((out_dim,), _full1),
        ]

    def head_args(p):
        return [p['ng'], p['nb'], p['w1'], p['b1'],
                p['w2'], p['b2'], p['wh'], p['bh']]

    in_specs = [
        pl.BlockSpec((B * K, EMB), _full),
        pl.BlockSpec((BLOCK_N, K), _row),
        pl.BlockSpec((BLOCK_N, 1), _row),
        pl.BlockSpec((1, B * K), _full),
        pl.BlockSpec((BLOCK_N, EMB), _row),
        pl.BlockSpec((BLOCK_N, 1), _row),
        pl.BlockSpec((BLOCK_N, 1), _row),
        pl.BlockSpec((BLOCK_N, 1), _row),
        pl.BlockSpec((BLOCK_N, 1), _row),
        pl.BlockSpec((EMB, EMB), _full),
        pl.BlockSpec((EMB,), _full1),
        pl.BlockSpec((EMB,), _full1),
        pl.BlockSpec((EMB,), _full1),
    ] + head_specs(1) + head_specs(NCLS) + head_specs(1)

    out_specs = [
        pl.BlockSpec((BLOCK_N, EMB), _row),
        pl.BlockSpec((BLOCK_N, 1), _row),
        pl.BlockSpec((BLOCK_N, NCLS), _row),
        pl.BlockSpec((BLOCK_N, 1), _row),
    ]
    out_shape = [
        jax.ShapeDtypeStruct((n, EMB), jnp.float32),
        jax.ShapeDtypeStruct((n, 1), jnp.float32),
        jax.ShapeDtypeStruct((n, NCLS), jnp.float32),
        jax.ShapeDtypeStruct((n, 1), jnp.float32),
    ]

    args = [evolved_tokens, attn_weights, vb2, colb, z_var_0,
            mbin, msmall, mlarge, ranges,
            params['deslice_w'], params['deslice_b'], params['fus_g'], params['fus_b']]
    args += head_args(params['bin']) + head_args(params['ints']) + head_args(params['intl'])

    z_out, prob_bin, logits_int_small, pred_int_large = pl.pallas_call(
        _fused_kernel,
        grid=grid,
        in_specs=in_specs,
        out_specs=out_specs,
        out_shape=out_shape,
        scratch_shapes=[
            pltpu.VMEM((B * K, EMB), jnp.bfloat16),
            pltpu.VMEM((EMB, 3 * EMB), jnp.bfloat16),
            pltpu.VMEM((1, 3 * EMB), jnp.bfloat16),
            pltpu.VMEM((EMB, 3 * EMB), jnp.bfloat16),
            pltpu.VMEM((1, 3 * EMB), jnp.bfloat16),
            pltpu.VMEM((EMB, 3 * 128), jnp.bfloat16),
        ],
    )(*args)
    return (z_out, prob_bin, logits_int_small, pred_int_large)
